# baseline (speedup 1.0000x reference)
import functools

import jax
import jax.numpy as jnp
from jax import lax
from jax.experimental import pallas as pl
from jax.experimental.pallas import tpu as pltpu
from jax.experimental.pallas import tpu_sc as plsc

B, S = 4096, 200
D = 64
V = 1000000
NFULL = V // 128
VTAIL = V - NFULL * 128

_INFO = plsc.get_sparse_core_info()
_NC, _NS = _INFO.num_cores, _INFO.num_subcores
_NW = _NC * _NS

_MESH = plsc.VectorSubcoreMesh(core_axis_name="c", subcore_axis_name="s")
_TILED = pltpu.CompilerParams(use_tc_tiling_on_sc=True,
                              needs_layout_passes=False)


def _iotas():
    base = jnp.arange(16, dtype=jnp.int32)
    return [base + 16 * k for k in range(8)]


@functools.partial(
    pl.kernel,
    mesh=_MESH,
    out_type=jax.ShapeDtypeStruct((V, 128), jnp.float32),
    scratch_types=[
        pltpu.VMEM((64, 128), jnp.float32),
        pltpu.VMEM((64, 128), jnp.float32),
        pltpu.VMEM((128, 128), jnp.float32),
        pltpu.VMEM((128, 128), jnp.float32),
        pltpu.VMEM((64, 64), jnp.float32),
        pltpu.VMEM((64, 128), jnp.float32),
        pltpu.SemaphoreType.DMA,
        pltpu.SemaphoreType.DMA,
        pltpu.SemaphoreType.DMA,
        pltpu.SemaphoreType.DMA,
    ],
    compiler_params=_TILED,
)
def _repack(wt_hbm, wd_hbm, s0, s1, d0, d1, s_t, d_t, rs0, rs1, ws0, ws1):
    wid = lax.axis_index("s") * _NC + lax.axis_index("c")
    cnt = jnp.where(wid < 2, 246, 244)
    base = jnp.where(wid < 2, 246 * wid, 492 + 244 * (wid - 2))
    n_it = cnt // 2
    rows = _iotas()[:4]
    zero16 = jnp.zeros((16,), jnp.int32)
    srcs = (s0, s1)
    dsts = (d0, d1)
    rsem = (rs0, rs1)
    wsem = (ws0, ws1)

    def fire_read(i, p):
        pltpu.async_copy(wt_hbm.at[:, pl.ds((base + i) * 128, 128)],
                         srcs[p], rsem[p])

    def wait_read(p):
        pltpu.make_async_copy(wt_hbm.at[:, pl.ds(0, 128)], srcs[p],
                              rsem[p]).wait()

    def fire_write(i, p):
        pltpu.async_copy(dsts[p], wd_hbm.at[pl.ds((base + i) * 128, 128)],
                         wsem[p])

    def wait_write(p):
        pltpu.make_async_copy(dsts[p], wd_hbm.at[pl.ds(0, 128)],
                              wsem[p]).wait()

    def pack(src, dst, nj):
        mask = nj - 1

        @plsc.parallel_loop(0, nj, 1, unroll=4)
        def _(d):
            for k in range(4):
                cvec = rows[k]
                jvec = lax.bitwise_and(d + cvec, mask)
                val = plsc.load_gather(src, [cvec, jvec])
                plsc.store_scatter(dst, [jvec, cvec], val)

    fire_read(0, 0)
    fire_read(1, 1)

    def body(it, carry):
        i = it * 2
        for p in range(2):
            wait_read(p)

            @pl.when(it > 0)
            def _():
                wait_write(p)
            pack(srcs[p], dsts[p], 128)
            fire_write(i + p, p)

            @pl.when(it + 1 < n_it)
            def _():
                fire_read(i + p + 2, p)
        return carry

    lax.fori_loop(0, n_it, body, 0)
    wait_write(0)
    wait_write(1)

    @pl.when(wid == _NW - 1)
    def _():
        pltpu.sync_copy(wt_hbm.at[:, pl.ds(NFULL * 128, VTAIL)], s_t)
        pack(s_t, d_t, VTAIL)
        pltpu.sync_copy(d_t, wd_hbm.at[pl.ds(NFULL * 128, VTAIL)])


@functools.partial(
    pl.kernel,
    mesh=_MESH,
    out_type=jax.ShapeDtypeStruct((B, S, D), jnp.float32),
    scratch_types=[
        pltpu.VMEM((128, S), jnp.int32),
        pltpu.VMEM((128,), jnp.int32),
        pltpu.VMEM((128,), jnp.int32),
        pltpu.VMEM((128, 128), jnp.float32),
        pltpu.VMEM((128, 128), jnp.float32),
        pltpu.VMEM((128, 64), jnp.float32),
        pltpu.VMEM((128, 64), jnp.float32),
        pltpu.SemaphoreType.DMA,
        pltpu.SemaphoreType.DMA,
        pltpu.SemaphoreType.DMA,
        pltpu.SemaphoreType.DMA,
    ],
    compiler_params=_TILED,
)
def _gather(wd_hbm, x_hbm, out_hbm, x_v, vi0, vi1, pr0, pr1, t0, t1,
            gs0, gs1, ss0, ss1):
    wid = lax.axis_index("s") * _NC + lax.axis_index("c")
    bpw = B // _NW
    b0 = wid * bpw
    rows = _iotas()
    zero16 = jnp.zeros((16,), jnp.int32)
    vidx = (vi0, vi1)
    wide = (pr0, pr1)
    tile = (t0, t1)
    gsem = (gs0, gs1)
    ssem = (ss0, ss1)

    pltpu.sync_copy(x_hbm.at[pl.ds(b0, bpw)], x_v)

    def extract_fire(s, p):
        scol = zero16 + s
        for k in range(8):
            r = plsc.load_gather(x_v, [rows[k], scol])
            vidx[p][pl.ds(16 * k, 16)] = r
        pltpu.async_copy(wd_hbm.at[vidx[p]], wide[p], gsem[p])

    def wait_gather(p):
        pltpu.make_async_copy(wd_hbm.at[pl.ds(0, 128)], wide[p],
                              gsem[p]).wait()

    def compact(p):
        wd, tl = wide[p], tile[p]

        @plsc.parallel_loop(0, 128, 1, unroll=4)
        def _(j):
            for k in range(4):
                tl[j, pl.ds(16 * k, 16)] = wd[j, pl.ds(16 * k, 16)]

    def fire_store(s, p):
        pltpu.async_copy(tile[p], out_hbm.at[pl.ds(b0, bpw), s], ssem[p])

    def wait_store(p):
        pltpu.make_async_copy(tile[p], out_hbm.at[pl.ds(0, bpw), 0],
                              ssem[p]).wait()

    extract_fire(0, 0)
    extract_fire(1, 1)

    def body(it, carry):
        s = it * 2
        for p in range(2):
            wait_gather(p)

            @pl.when(it > 0)
            def _():
                wait_store(p)
            compact(p)
            fire_store(s + p, p)

            @pl.when(it + 1 < S // 2)
            def _():
                extract_fire(s + p + 2, p)
        return carry

    lax.fori_loop(0, S // 2, body, 0)
    wait_store(0)
    wait_store(1)


def kernel(x, weight):
    w_dup = _repack(weight.T)
    return _gather(w_dup, x.astype(jnp.int32))


# speedup vs baseline: 1.5866x; 1.5866x over previous
import functools

import jax
import jax.numpy as jnp
from jax import lax
from jax.experimental import pallas as pl
from jax.experimental.pallas import tpu as pltpu
from jax.experimental.pallas import tpu_sc as plsc

B, S = 4096, 200
D = 64
V = 1000000
NFULL = V // 128
VTAIL = V - NFULL * 128

_INFO = plsc.get_sparse_core_info()
_NC, _NS = _INFO.num_cores, _INFO.num_subcores
_NW = _NC * _NS

_MESH = plsc.VectorSubcoreMesh(core_axis_name="c", subcore_axis_name="s")
_TILED = pltpu.CompilerParams(use_tc_tiling_on_sc=True,
                              needs_layout_passes=False)


def _iotas():
    base = jnp.arange(16, dtype=jnp.int32)
    return [base + 16 * k for k in range(8)]


@functools.partial(
    pl.kernel,
    mesh=_MESH,
    out_type=jax.ShapeDtypeStruct((V, 128), jnp.float32),
    scratch_types=[
        pltpu.VMEM((64, 128), jnp.float32),
        pltpu.VMEM((64, 128), jnp.float32),
        pltpu.VMEM((128, 128), jnp.float32),
        pltpu.VMEM((128, 128), jnp.float32),
        pltpu.VMEM((64, 64), jnp.float32),
        pltpu.VMEM((64, 128), jnp.float32),
        pltpu.SemaphoreType.DMA,
        pltpu.SemaphoreType.DMA,
        pltpu.SemaphoreType.DMA,
        pltpu.SemaphoreType.DMA,
    ],
    compiler_params=_TILED,
)
def _repack(wt_hbm, wd_hbm, s0, s1, d0, d1, s_t, d_t, rs0, rs1, ws0, ws1):
    wid = lax.axis_index("s") * _NC + lax.axis_index("c")
    cnt = jnp.where(wid < 2, 246, 244)
    base = jnp.where(wid < 2, 246 * wid, 492 + 244 * (wid - 2))
    n_it = cnt // 2
    rows = _iotas()[:4]
    zero16 = jnp.zeros((16,), jnp.int32)
    srcs = (s0, s1)
    dsts = (d0, d1)
    rsem = (rs0, rs1)
    wsem = (ws0, ws1)

    def fire_read(i, p):
        pltpu.async_copy(wt_hbm.at[:, pl.ds((base + i) * 128, 128)],
                         srcs[p], rsem[p])

    def wait_read(p):
        pltpu.make_async_copy(wt_hbm.at[:, pl.ds(0, 128)], srcs[p],
                              rsem[p]).wait()

    def fire_write(i, p):
        pltpu.async_copy(dsts[p], wd_hbm.at[pl.ds((base + i) * 128, 128)],
                         wsem[p])

    def wait_write(p):
        pltpu.make_async_copy(dsts[p], wd_hbm.at[pl.ds(0, 128)],
                              wsem[p]).wait()

    def pack(src, dst, nj):
        mask = nj - 1

        @plsc.parallel_loop(0, nj, 1, unroll=4)
        def _(d):
            for k in range(4):
                cvec = rows[k]
                jvec = lax.bitwise_and(d + cvec, mask)
                val = plsc.load_gather(src, [cvec, jvec])
                plsc.store_scatter(dst, [jvec, cvec], val)

    fire_read(0, 0)
    fire_read(1, 1)

    def body(it, carry):
        i = it * 2
        for p in range(2):
            wait_read(p)

            @pl.when(it > 0)
            def _():
                wait_write(p)
            pack(srcs[p], dsts[p], 128)
            fire_write(i + p, p)

            @pl.when(it + 1 < n_it)
            def _():
                fire_read(i + p + 2, p)
        return carry

    lax.fori_loop(0, n_it, body, 0)
    wait_write(0)
    wait_write(1)

    @pl.when(wid == _NW - 1)
    def _():
        pltpu.sync_copy(wt_hbm.at[:, pl.ds(NFULL * 128, VTAIL)], s_t)
        pack(s_t, d_t, VTAIL)
        pltpu.sync_copy(d_t, wd_hbm.at[pl.ds(NFULL * 128, VTAIL)])


@functools.partial(
    pl.kernel,
    mesh=_MESH,
    out_type=jax.ShapeDtypeStruct((S, D, B), jnp.float32),
    scratch_types=[
        pltpu.VMEM((128, S), jnp.int32),
        pltpu.VMEM((128,), jnp.int32),
        pltpu.VMEM((128,), jnp.int32),
        pltpu.VMEM((128, 128), jnp.float32),
        pltpu.VMEM((128, 128), jnp.float32),
        pltpu.VMEM((64, 128), jnp.float32),
        pltpu.VMEM((64, 128), jnp.float32),
        pltpu.SemaphoreType.DMA,
        pltpu.SemaphoreType.DMA,
        pltpu.SemaphoreType.DMA,
        pltpu.SemaphoreType.DMA,
    ],
    compiler_params=_TILED,
)
def _gather(wd_hbm, x_hbm, out_hbm, x_v, vi0, vi1, pr0, pr1, t0, t1,
            gs0, gs1, ss0, ss1):
    wid = lax.axis_index("s") * _NC + lax.axis_index("c")
    bpw = B // _NW
    b0 = wid * bpw
    rows = _iotas()
    zero16 = jnp.zeros((16,), jnp.int32)
    vidx = (vi0, vi1)
    wide = (pr0, pr1)
    tile = (t0, t1)
    gsem = (gs0, gs1)
    ssem = (ss0, ss1)

    pltpu.sync_copy(x_hbm.at[pl.ds(b0, bpw)], x_v)

    def extract_fire(s, p):
        scol = zero16 + s
        for k in range(8):
            r = plsc.load_gather(x_v, [rows[k], scol])
            vidx[p][pl.ds(16 * k, 16)] = r
        pltpu.async_copy(wd_hbm.at[vidx[p]], wide[p], gsem[p])

    def wait_gather(p):
        pltpu.make_async_copy(wd_hbm.at[pl.ds(0, 128)], wide[p],
                              gsem[p]).wait()

    def compact(p):
        wd, tl = wide[p], tile[p]

        @plsc.parallel_loop(0, 128, 1, unroll=4)
        def _(d):
            for k in range(4):
                cvec = rows[k]
                jvec = lax.bitwise_and(d + cvec, 127)
                val = plsc.load_gather(wd, [jvec, cvec])
                plsc.store_scatter(tl, [cvec, jvec], val)

    def fire_store(s, p):
        pltpu.async_copy(tile[p], out_hbm.at[s, :, pl.ds(b0, bpw)], ssem[p])

    def wait_store(p):
        pltpu.make_async_copy(tile[p], out_hbm.at[0, :, pl.ds(0, bpw)],
                              ssem[p]).wait()

    extract_fire(0, 0)
    extract_fire(1, 1)

    def body(it, carry):
        s = it * 2
        for p in range(2):
            wait_gather(p)

            @pl.when(it > 0)
            def _():
                wait_store(p)
            compact(p)
            fire_store(s + p, p)

            @pl.when(it + 1 < S // 2)
            def _():
                extract_fire(s + p + 2, p)
        return carry

    lax.fori_loop(0, S // 2, body, 0)
    wait_store(0)
    wait_store(1)


def kernel(x, weight):
    w_dup = _repack(weight.T)
    out_t = _gather(w_dup, x.astype(jnp.int32))
    return out_t.transpose(2, 0, 1)
